# Initial kernel scaffold; baseline (speedup 1.0000x reference)
import jax
import jax.numpy as jnp
from jax.experimental import pallas as pl


def kernel(x, edge_index, W1, b1, W2, b2):
    raise NotImplementedError("write your pallas kernel here")


# speedup vs baseline: 5.1102x; 5.1102x over previous
import functools

import jax
import jax.numpy as jnp
from jax import lax
from jax.experimental import pallas as pl
from jax.experimental.pallas import tpu as pltpu
from jax.experimental.pallas import tpu_sc as plsc

N_NODES = 10000
N_EDGES = 320000
F_IN = 128
F_HID = 512

NC = 2
NS = 16
NW = NC * NS
CHUNK = 128
CPW = -(-N_EDGES // (NW * CHUNK))
EW = CPW * CHUNK
E_PAD = EW * NW
N_PAD = 10240
RPS = N_PAD // NS

BLK = 1024
GRID = N_PAD // BLK


_mesh = plsc.VectorSubcoreMesh(core_axis_name="c", subcore_axis_name="s")


def _f32(shape):
    return jax.ShapeDtypeStruct(shape, jnp.float32)


def _degrees(src, dst, zrows, ones_rows):
    @functools.partial(
        pl.kernel,
        out_type=[_f32((NC, N_PAD, F_IN)), _f32((NC, N_PAD, F_IN))],
        mesh=_mesh,
        scratch_types=[
            pltpu.VMEM_SHARED((N_PAD, F_IN), jnp.float32),
            pltpu.VMEM((CHUNK, F_IN), jnp.float32),
            pltpu.VMEM((CHUNK,), jnp.int32),
        ],
    )
    def k(src_h, dst_h, z_h, ones_h, dego_h, degi_h, acc_sh, ones_v, idx_v):
        c = lax.axis_index("c")
        s = lax.axis_index("s")
        base = (c * NS + s) * EW
        rbase = s * RPS
        pltpu.sync_copy(ones_h, ones_v)
        for in_ref, out_ref in ((src_h, dego_h), (dst_h, degi_h)):
            pltpu.sync_copy(z_h, acc_sh.at[pl.ds(rbase, RPS)])
            plsc.subcore_barrier()

            @pl.loop(0, CPW)
            def _(j, in_ref=in_ref):
                pltpu.sync_copy(in_ref.at[pl.ds(base + j * CHUNK, CHUNK)], idx_v)
                pltpu.sync_copy(ones_v, acc_sh.at[idx_v], add=True)

            plsc.subcore_barrier()
            pltpu.sync_copy(acc_sh.at[pl.ds(rbase, RPS)],
                            out_ref.at[c, pl.ds(rbase, RPS)])

    return k(src, dst, zrows, ones_rows)


def _propagate(table, src, dst, zrows):
    @functools.partial(
        pl.kernel,
        out_type=_f32((NC, N_PAD, F_IN)),
        mesh=_mesh,
        scratch_types=[
            pltpu.VMEM_SHARED((N_PAD, F_IN), jnp.float32),
            pltpu.VMEM((CHUNK, F_IN), jnp.float32),
            pltpu.VMEM((CHUNK,), jnp.int32),
            pltpu.VMEM((CHUNK,), jnp.int32),
            pltpu.SemaphoreType.DMA,
        ],
    )
    def k(tab_h, src_h, dst_h, z_h, out_h, acc_sh, rows_v, sidx, didx, sem):
        c = lax.axis_index("c")
        s = lax.axis_index("s")
        base = (c * NS + s) * EW
        rbase = s * RPS
        pltpu.sync_copy(z_h, acc_sh.at[pl.ds(rbase, RPS)])
        plsc.subcore_barrier()

        @pl.loop(0, CPW)
        def _(j):
            pltpu.sync_copy(src_h.at[pl.ds(base + j * CHUNK, CHUNK)], sidx)
            pltpu.sync_copy(dst_h.at[pl.ds(base + j * CHUNK, CHUNK)], didx)
            pltpu.async_copy(tab_h.at[sidx], rows_v, sem).wait()
            pltpu.sync_copy(rows_v, acc_sh.at[didx], add=True)

        plsc.subcore_barrier()
        pltpu.sync_copy(acc_sh.at[pl.ds(rbase, RPS)], out_h.at[c, pl.ds(rbase, RPS)])

    return k(table, src, dst, zrows)


def _norm_from_deg(deg):
    return jnp.where(deg > 0, lax.rsqrt(jnp.maximum(deg, 1.0)), 0.0)


def _norm_h0_body(x_ref, dego_ref, degi_ref, h0_ref, ns_ref, nd_ref):
    deg_o = dego_ref[0] + dego_ref[1]
    deg_i = degi_ref[0] + degi_ref[1]
    row = lax.broadcasted_iota(jnp.int32, (N_PAD, 1), 0)
    valid = (row < N_NODES).astype(jnp.float32)
    ns = _norm_from_deg(deg_o) * valid
    nd = _norm_from_deg(deg_i) * valid
    ns_ref[...] = ns
    nd_ref[...] = nd
    h0_ref[...] = x_ref[...] * ns


def _norm_h0(x_pad, dego, degi):
    return pl.pallas_call(
        _norm_h0_body,
        out_shape=[_f32((N_PAD, F_IN)), _f32((N_PAD, 1)), _f32((N_PAD, 1))],
    )(x_pad, dego, degi)


def _mm_body(p_ref, ns_ref, nd_ref, w1_ref, b1_ref, w2_ref, g_ref):
    p = (p_ref[0] + p_ref[1]) * nd_ref[...]
    h1 = jnp.dot(p, w1_ref[...], preferred_element_type=jnp.float32,
                 precision=lax.Precision.HIGHEST)
    h1 = jnp.maximum(h1 + b1_ref[...], 0.0) * ns_ref[...]
    g_ref[...] = jnp.dot(h1, w2_ref[...], preferred_element_type=jnp.float32,
                         precision=lax.Precision.HIGHEST)


def _mm(p1, ns, nd, W1, b1, W2):
    return pl.pallas_call(
        _mm_body,
        grid=(GRID,),
        in_specs=[
            pl.BlockSpec((NC, BLK, F_IN), lambda i: (0, i, 0)),
            pl.BlockSpec((BLK, 1), lambda i: (i, 0)),
            pl.BlockSpec((BLK, 1), lambda i: (i, 0)),
            pl.BlockSpec((F_IN, F_HID), lambda i: (0, 0)),
            pl.BlockSpec((1, F_HID), lambda i: (0, 0)),
            pl.BlockSpec((F_HID, F_IN), lambda i: (0, 0)),
        ],
        out_specs=pl.BlockSpec((BLK, F_IN), lambda i: (i, 0)),
        out_shape=_f32((N_PAD, F_IN)),
    )(p1, ns, nd, W1, b1, W2)


def _fin_body(q_ref, nd_ref, b2_ref, o_ref):
    o_ref[...] = (q_ref[0] + q_ref[1]) * nd_ref[...] + b2_ref[...]


def _fin(p2, nd, b2):
    return pl.pallas_call(
        _fin_body,
        grid=(GRID,),
        in_specs=[
            pl.BlockSpec((NC, BLK, F_IN), lambda i: (0, i, 0)),
            pl.BlockSpec((BLK, 1), lambda i: (i, 0)),
            pl.BlockSpec((1, F_IN), lambda i: (0, 0)),
        ],
        out_specs=pl.BlockSpec((BLK, F_IN), lambda i: (i, 0)),
        out_shape=_f32((N_PAD, F_IN)),
    )(p2, nd, b2)


@jax.jit
def kernel(x, edge_index, W1, b1, W2, b2):
    src = edge_index[0].astype(jnp.int32)
    dst = edge_index[1].astype(jnp.int32)
    pad = jnp.full((E_PAD - N_EDGES,), N_NODES, jnp.int32)
    src_p = jnp.concatenate([src, pad])
    dst_p = jnp.concatenate([dst, pad])
    x_p = jnp.pad(x, ((0, N_PAD - N_NODES), (0, 0)))

    ones_rows = jnp.ones((CHUNK, F_IN), jnp.float32)
    zfeat = jnp.zeros((RPS, F_IN), jnp.float32)

    dego, degi = _degrees(src_p, dst_p, zfeat, ones_rows)
    h0, ns, nd = _norm_h0(x_p, dego[:, :, 0:1], degi[:, :, 0:1])
    p1 = _propagate(h0, src_p, dst_p, zfeat)
    g = _mm(p1, ns, nd, W1, b1.reshape(1, F_HID), W2)
    p2 = _propagate(g, src_p, dst_p, zfeat)
    out = _fin(p2, nd, b2.reshape(1, F_IN))
    return out[:N_NODES]
